# Initial kernel scaffold; baseline (speedup 1.0000x reference)
import jax
import jax.numpy as jnp
from jax.experimental import pallas as pl


def kernel(x, edge_index, edge_attr, current_node_idx, reachable_indices, neighbor_edge_idx, W1, b1, W2, b2, Wm1, bm1, Wm2, bm2):
    raise NotImplementedError("write your pallas kernel here")


# speedup vs baseline: 24.6159x; 24.6159x over previous
import functools

import jax
import jax.numpy as jnp
from jax import lax
from jax.experimental import pallas as pl
from jax.experimental.pallas import tpu as pltpu
from jax.experimental.pallas import tpu_sc as plsc

N = 10000
NP = 10240
E = 320000
D = 128
H = 64
DE = 16
K = 64

NC = 2
NS = 16
NW = NC * NS
EPW = E // NW
CHUNK = 1000
NCHUNK = EPW // CHUNK
ROWS_PER_TILE = NP // NS

_mesh = plsc.VectorSubcoreMesh(
    core_axis_name="c", subcore_axis_name="s", num_cores=NC, num_subcores=NS)


@functools.partial(
    pl.kernel,
    out_type=jax.ShapeDtypeStruct((NC, NP, 16), jnp.float32),
    mesh=_mesh,
    compiler_params=pltpu.CompilerParams(use_tc_tiling_on_sc=False),
    scratch_types=[
        pltpu.VMEM((CHUNK,), jnp.int32),
        pltpu.VMEM((CHUNK, 16), jnp.float32),
        pltpu.VMEM_SHARED((NP, 16), jnp.float32),
        pltpu.SemaphoreType.DMA,
    ],
)
def _sc_degree(dst1d, ones_t, zeros16, out, dstv, onesv, acc, sem):
    c = lax.axis_index("c")
    s = lax.axis_index("s")
    g = c * NS + s

    pltpu.sync_copy(zeros16, acc.at[pl.ds(s * ROWS_PER_TILE, ROWS_PER_TILE)])
    pltpu.sync_copy(ones_t, onesv)
    plsc.subcore_barrier()

    for b in range(NCHUNK):
        e0 = g * EPW + b * CHUNK
        pltpu.sync_copy(dst1d.at[pl.ds(e0, CHUNK)], dstv)
        pltpu.sync_copy(onesv, acc.at[dstv], add=True)

    plsc.subcore_barrier()
    pltpu.sync_copy(acc.at[pl.ds(s * ROWS_PER_TILE, ROWS_PER_TILE)],
                    out.at[c, pl.ds(s * ROWS_PER_TILE, ROWS_PER_TILE)])


@functools.partial(
    pl.kernel,
    out_type=jax.ShapeDtypeStruct((NC, NP, H), jnp.float32),
    mesh=_mesh,
    compiler_params=pltpu.CompilerParams(use_tc_tiling_on_sc=False),
    scratch_types=[
        pltpu.VMEM((CHUNK,), jnp.int32),
        pltpu.VMEM((CHUNK,), jnp.int32),
        pltpu.VMEM((CHUNK, H), jnp.float32),
        pltpu.VMEM_SHARED((NP, H), jnp.float32),
        pltpu.SemaphoreType.DMA,
    ],
)
def _sc_segsum(src1d, dst1d, table, zeros64, out, srcv, dstv, rowbuf, acc, sem):
    c = lax.axis_index("c")
    s = lax.axis_index("s")
    g = c * NS + s

    pltpu.sync_copy(zeros64, acc.at[pl.ds(s * ROWS_PER_TILE, ROWS_PER_TILE)])
    plsc.subcore_barrier()

    for b in range(NCHUNK):
        e0 = g * EPW + b * CHUNK
        pltpu.sync_copy(src1d.at[pl.ds(e0, CHUNK)], srcv)
        pltpu.sync_copy(dst1d.at[pl.ds(e0, CHUNK)], dstv)
        pltpu.async_copy(table.at[srcv], rowbuf, sem).wait()
        pltpu.sync_copy(rowbuf, acc.at[dstv], add=True)

    plsc.subcore_barrier()
    pltpu.sync_copy(acc.at[pl.ds(s * ROWS_PER_TILE, ROWS_PER_TILE)],
                    out.at[c, pl.ds(s * ROWS_PER_TILE, ROWS_PER_TILE)])


def _tc_edges_body(ei_ref, src_ref, dst_ref):
    src_ref[...] = ei_ref[0]
    dst_ref[...] = ei_ref[1]


def _tc_edges(ei):
    return pl.pallas_call(
        _tc_edges_body,
        out_shape=(jax.ShapeDtypeStruct((E,), jnp.int32),
                   jax.ShapeDtypeStruct((E,), jnp.int32)),
    )(ei)


def _tc_layer1_body(x_ref, w1_ref, degp_ref, z1_ref, norm_ref):
    deg = degp_ref[0, :, 0:1] + degp_ref[1, :, 0:1] + 1.0
    norm = lax.rsqrt(deg)
    u = jnp.dot(x_ref[...], w1_ref[...], preferred_element_type=jnp.float32)
    z1_ref[0:N] = u * norm[0:N]
    z1_ref[N:NP] = jnp.zeros((NP - N, H), jnp.float32)
    norm_ref[...] = norm


def _tc_layer1(x, w1, degparts):
    return pl.pallas_call(
        _tc_layer1_body,
        out_shape=(jax.ShapeDtypeStruct((NP, H), jnp.float32),
                   jax.ShapeDtypeStruct((NP, 1), jnp.float32)),
    )(x, w1, degparts)


def _tc_layer2_body(aggp_ref, z1_ref, norm_ref, w2_ref, b1_ref, z2_ref):
    norm = norm_ref[...]
    ssum = aggp_ref[0] + aggp_ref[1] + z1_ref[...]
    h = jnp.maximum(norm * ssum + b1_ref[...], 0.0)
    z2_ref[...] = jnp.dot(h, w2_ref[...],
                          preferred_element_type=jnp.float32) * norm


def _tc_layer2(aggparts, z1, norm, w2, b1r):
    return pl.pallas_call(
        _tc_layer2_body,
        out_shape=jax.ShapeDtypeStruct((NP, H), jnp.float32),
    )(aggparts, z1, norm, w2, b1r)


def _tc_final_body(aggp_ref, z2_ref, norm_ref, b2_ref, wa_ref, wb_ref,
                   wc_ref, bm1_ref, wm2_ref, bm2_ref, ridx_ref, cidx_ref,
                   eidx_ref, ea_hbm, q_ref, h2_ref, hn_ref, ea_ref, sem):
    norm = norm_ref[...]
    ssum = aggp_ref[0] + aggp_ref[1] + z2_ref[...]
    h2_ref[...] = jnp.maximum(norm * ssum + b2_ref[...], 0.0)

    def gather_body(k, _):
        idx = ridx_ref[k]
        hn_ref[pl.ds(k, 1), :] = h2_ref[pl.ds(idx, 1), :]
        eidx = eidx_ref[k]
        pltpu.make_async_copy(ea_hbm.at[pl.ds(eidx, 1)],
                              ea_ref.at[pl.ds(k, 1)], sem).start()
        return 0
    lax.fori_loop(0, K, gather_body, 0)

    def drain_body(k, _):
        pltpu.make_async_copy(ea_hbm.at[pl.ds(0, 1)],
                              ea_ref.at[pl.ds(0, 1)], sem).wait()
        return 0
    lax.fori_loop(0, K, drain_body, 0)

    cur = cidx_ref[0]
    hcur = h2_ref[pl.ds(cur, 1), :]
    acc = jnp.dot(hcur, wa_ref[...], preferred_element_type=jnp.float32)
    acc = acc + jnp.dot(hn_ref[...], wb_ref[...],
                        preferred_element_type=jnp.float32)
    acc = acc + jnp.dot(ea_ref[...], wc_ref[...],
                        preferred_element_type=jnp.float32)
    pre = jnp.maximum(acc + bm1_ref[...], 0.0)
    q = jnp.sum(pre * wm2_ref[...], axis=1, keepdims=True) + bm2_ref[...]
    q_ref[...] = q


def _tc_final(aggparts, z2, norm, b2r, wa, wb, wc, bm1r, wm2r, bm2r,
              ridx, cidx, eidx, edge_attr):
    return pl.pallas_call(
        _tc_final_body,
        out_shape=jax.ShapeDtypeStruct((K, 1), jnp.float32),
        in_specs=[pl.BlockSpec(memory_space=pl.ANY)
                  if i == 13 else
                  (pl.BlockSpec(memory_space=pltpu.SMEM)
                   if i in (10, 11, 12) else pl.BlockSpec(memory_space=pltpu.VMEM))
                  for i in range(14)],
        out_specs=pl.BlockSpec(memory_space=pltpu.VMEM),
        scratch_shapes=[
            pltpu.VMEM((NP, H), jnp.float32),
            pltpu.VMEM((K, H), jnp.float32),
            pltpu.VMEM((K, DE), jnp.float32),
            pltpu.SemaphoreType.DMA,
        ],
    )(aggparts, z2, norm, b2r, wa, wb, wc, bm1r, wm2r, bm2r,
      ridx, cidx, eidx, edge_attr)


def kernel(x, edge_index, edge_attr, current_node_idx, reachable_indices,
           neighbor_edge_idx, W1, b1, W2, b2, Wm1, bm1, Wm2, bm2):
    src1d, dst1d = _tc_edges(edge_index.astype(jnp.int32))

    ones_t = jnp.ones((CHUNK, 16), jnp.float32)
    zeros16 = jnp.zeros((ROWS_PER_TILE, 16), jnp.float32)
    zeros64 = jnp.zeros((ROWS_PER_TILE, H), jnp.float32)

    degparts = _sc_degree(dst1d, ones_t, zeros16)
    z1, norm = _tc_layer1(x, W1, degparts)
    agg1 = _sc_segsum(src1d, dst1d, z1, zeros64)
    z2 = _tc_layer2(agg1, z1, norm, W2, b1.reshape(1, H))
    agg2 = _sc_segsum(src1d, dst1d, z2, zeros64)

    ridx = reachable_indices.astype(jnp.int32)
    cidx = jnp.asarray(current_node_idx, jnp.int32).reshape(1)
    eidx = neighbor_edge_idx.astype(jnp.int32)
    wa = Wm1[:H]
    wb = Wm1[H:2 * H]
    wc = Wm1[2 * H:]
    q = _tc_final(agg2, z2, norm, b2.reshape(1, H), wa, wb, wc,
                  bm1.reshape(1, 2 * H), Wm2.reshape(1, 2 * H),
                  bm2.reshape(1, 1), ridx, cidx, eidx, edge_attr)
    return q.reshape(K)


# speedup vs baseline: 26.9927x; 1.0966x over previous
import functools

import jax
import jax.numpy as jnp
from jax import lax
from jax.experimental import pallas as pl
from jax.experimental.pallas import tpu as pltpu
from jax.experimental.pallas import tpu_sc as plsc

N = 10000
NP = 10240
E = 320000
D = 128
H = 64
DE = 16
K = 64

NC = 2
NS = 16
NW = NC * NS
EPW = E // NW
CH = 400
NCH = EPW // CH
CHD = 2000
NCHD = EPW // CHD
ROWS_PER_TILE = NP // NS

_mesh = plsc.VectorSubcoreMesh(
    core_axis_name="c", subcore_axis_name="s", num_cores=NC, num_subcores=NS)


@functools.partial(
    pl.kernel,
    out_type=jax.ShapeDtypeStruct((NC, NP, 16), jnp.float32),
    mesh=_mesh,
    compiler_params=pltpu.CompilerParams(use_tc_tiling_on_sc=False),
    scratch_types=[
        pltpu.VMEM((EPW,), jnp.int32),
        pltpu.VMEM((CHD, 16), jnp.float32),
        pltpu.VMEM_SHARED((NP, 16), jnp.float32),
        pltpu.SemaphoreType.DMA,
        pltpu.SemaphoreType.DMA,
    ],
)
def _sc_degree(dst1d, ones_t, zeros16, out, dstv, onesv, acc, semi, sems):
    c = lax.axis_index("c")
    s = lax.axis_index("s")
    g = c * NS + s

    ci = pltpu.async_copy(dst1d.at[pl.ds(g * EPW, EPW)], dstv, semi)
    pltpu.sync_copy(zeros16, acc.at[pl.ds(s * ROWS_PER_TILE, ROWS_PER_TILE)])
    pltpu.sync_copy(ones_t, onesv)
    ci.wait()
    plsc.subcore_barrier()

    descs = [pltpu.async_copy(onesv, acc.at[dstv.at[pl.ds(b * CHD, CHD)]],
                              sems, add=True)
             for b in range(NCHD)]
    for d in descs:
        d.wait()

    plsc.subcore_barrier()
    pltpu.sync_copy(acc.at[pl.ds(s * ROWS_PER_TILE, ROWS_PER_TILE)],
                    out.at[c, pl.ds(s * ROWS_PER_TILE, ROWS_PER_TILE)])


@functools.partial(
    pl.kernel,
    out_type=jax.ShapeDtypeStruct((NC, NP, H), jnp.float32),
    mesh=_mesh,
    compiler_params=pltpu.CompilerParams(use_tc_tiling_on_sc=False),
    scratch_types=[
        pltpu.VMEM((EPW,), jnp.int32),
        pltpu.VMEM((EPW,), jnp.int32),
        pltpu.VMEM((CH, H), jnp.float32),
        pltpu.VMEM((CH, H), jnp.float32),
        pltpu.VMEM_SHARED((NP, H), jnp.float32),
        pltpu.SemaphoreType.DMA,
        pltpu.SemaphoreType.DMA,
        pltpu.SemaphoreType.DMA,
        pltpu.SemaphoreType.DMA,
        pltpu.SemaphoreType.DMA,
    ],
)
def _sc_segsum(src1d, dst1d, table, zeros64, out, srcv, dstv, rb0, rb1, acc,
               semi, semg0, semg1, sems0, sems1):
    c = lax.axis_index("c")
    s = lax.axis_index("s")
    g = c * NS + s
    e0 = g * EPW

    ci = pltpu.async_copy(src1d.at[pl.ds(e0, EPW)], srcv, semi)
    cd = pltpu.async_copy(dst1d.at[pl.ds(e0, EPW)], dstv, semi)
    pltpu.sync_copy(zeros64, acc.at[pl.ds(s * ROWS_PER_TILE, ROWS_PER_TILE)])
    ci.wait()
    cd.wait()
    plsc.subcore_barrier()

    rbs = (rb0, rb1)
    semg = (semg0, semg1)
    sems = (sems0, sems1)
    gd = [None, None]
    sd = [None, None]
    gd[0] = pltpu.async_copy(table.at[srcv.at[pl.ds(0, CH)]], rbs[0], semg[0])
    for b in range(NCH):
        sl = b % 2
        gd[sl].wait()
        sd[sl] = pltpu.async_copy(rbs[sl], acc.at[dstv.at[pl.ds(b * CH, CH)]],
                                  sems[sl], add=True)
        if b + 1 < NCH:
            ns = (b + 1) % 2
            if sd[ns] is not None:
                sd[ns].wait()
            gd[ns] = pltpu.async_copy(
                table.at[srcv.at[pl.ds((b + 1) * CH, CH)]], rbs[ns], semg[ns])
    sd[(NCH - 2) % 2].wait()
    sd[(NCH - 1) % 2].wait()

    plsc.subcore_barrier()
    pltpu.sync_copy(acc.at[pl.ds(s * ROWS_PER_TILE, ROWS_PER_TILE)],
                    out.at[c, pl.ds(s * ROWS_PER_TILE, ROWS_PER_TILE)])


def _tc_edges_body(ei_ref, src_ref, dst_ref):
    src_ref[...] = ei_ref[0]
    dst_ref[...] = ei_ref[1]


def _tc_edges(ei):
    return pl.pallas_call(
        _tc_edges_body,
        out_shape=(jax.ShapeDtypeStruct((E,), jnp.int32),
                   jax.ShapeDtypeStruct((E,), jnp.int32)),
    )(ei)


def _tc_layer1_body(x_ref, w1_ref, degp_ref, z1_ref, norm_ref):
    deg = degp_ref[0, :, 0:1] + degp_ref[1, :, 0:1] + 1.0
    norm = lax.rsqrt(deg)
    u = jnp.dot(x_ref[...], w1_ref[...], preferred_element_type=jnp.float32)
    z1_ref[0:N] = u * norm[0:N]
    z1_ref[N:NP] = jnp.zeros((NP - N, H), jnp.float32)
    norm_ref[...] = norm


def _tc_layer1(x, w1, degparts):
    return pl.pallas_call(
        _tc_layer1_body,
        out_shape=(jax.ShapeDtypeStruct((NP, H), jnp.float32),
                   jax.ShapeDtypeStruct((NP, 1), jnp.float32)),
    )(x, w1, degparts)


def _tc_layer2_body(aggp_ref, z1_ref, norm_ref, w2_ref, b1_ref, z2_ref):
    norm = norm_ref[...]
    ssum = aggp_ref[0] + aggp_ref[1] + z1_ref[...]
    h = jnp.maximum(norm * ssum + b1_ref[...], 0.0)
    z2_ref[...] = jnp.dot(h, w2_ref[...],
                          preferred_element_type=jnp.float32) * norm


def _tc_layer2(aggparts, z1, norm, w2, b1r):
    return pl.pallas_call(
        _tc_layer2_body,
        out_shape=jax.ShapeDtypeStruct((NP, H), jnp.float32),
    )(aggparts, z1, norm, w2, b1r)


def _tc_final_body(aggp_ref, z2_ref, norm_ref, b2_ref, wa_ref, wb_ref,
                   wc_ref, bm1_ref, wm2_ref, bm2_ref, ridx_ref, cidx_ref,
                   eidx_ref, ea_hbm, q_ref, h2_ref, hn_ref, ea_ref, sem):
    norm = norm_ref[...]
    ssum = aggp_ref[0] + aggp_ref[1] + z2_ref[...]
    h2_ref[...] = jnp.maximum(norm * ssum + b2_ref[...], 0.0)

    def gather_body(k, _):
        idx = ridx_ref[k]
        hn_ref[pl.ds(k, 1), :] = h2_ref[pl.ds(idx, 1), :]
        eidx = eidx_ref[k]
        pltpu.make_async_copy(ea_hbm.at[pl.ds(eidx, 1)],
                              ea_ref.at[pl.ds(k, 1)], sem).start()
        return 0
    lax.fori_loop(0, K, gather_body, 0)

    def drain_body(k, _):
        pltpu.make_async_copy(ea_hbm.at[pl.ds(0, 1)],
                              ea_ref.at[pl.ds(0, 1)], sem).wait()
        return 0
    lax.fori_loop(0, K, drain_body, 0)

    cur = cidx_ref[0]
    hcur = h2_ref[pl.ds(cur, 1), :]
    acc = jnp.dot(hcur, wa_ref[...], preferred_element_type=jnp.float32)
    acc = acc + jnp.dot(hn_ref[...], wb_ref[...],
                        preferred_element_type=jnp.float32)
    acc = acc + jnp.dot(ea_ref[...], wc_ref[...],
                        preferred_element_type=jnp.float32)
    pre = jnp.maximum(acc + bm1_ref[...], 0.0)
    q = jnp.sum(pre * wm2_ref[...], axis=1, keepdims=True) + bm2_ref[...]
    q_ref[...] = q


def _tc_final(aggparts, z2, norm, b2r, wa, wb, wc, bm1r, wm2r, bm2r,
              ridx, cidx, eidx, edge_attr):
    return pl.pallas_call(
        _tc_final_body,
        out_shape=jax.ShapeDtypeStruct((K, 1), jnp.float32),
        in_specs=[pl.BlockSpec(memory_space=pl.ANY)
                  if i == 13 else
                  (pl.BlockSpec(memory_space=pltpu.SMEM)
                   if i in (10, 11, 12) else pl.BlockSpec(memory_space=pltpu.VMEM))
                  for i in range(14)],
        out_specs=pl.BlockSpec(memory_space=pltpu.VMEM),
        scratch_shapes=[
            pltpu.VMEM((NP, H), jnp.float32),
            pltpu.VMEM((K, H), jnp.float32),
            pltpu.VMEM((K, DE), jnp.float32),
            pltpu.SemaphoreType.DMA,
        ],
    )(aggparts, z2, norm, b2r, wa, wb, wc, bm1r, wm2r, bm2r,
      ridx, cidx, eidx, edge_attr)


def kernel(x, edge_index, edge_attr, current_node_idx, reachable_indices,
           neighbor_edge_idx, W1, b1, W2, b2, Wm1, bm1, Wm2, bm2):
    src1d, dst1d = _tc_edges(edge_index.astype(jnp.int32))

    ones_t = jnp.ones((CHD, 16), jnp.float32)
    zeros16 = jnp.zeros((ROWS_PER_TILE, 16), jnp.float32)
    zeros64 = jnp.zeros((ROWS_PER_TILE, H), jnp.float32)

    degparts = _sc_degree(dst1d, ones_t, zeros16)
    z1, norm = _tc_layer1(x, W1, degparts)
    agg1 = _sc_segsum(src1d, dst1d, z1, zeros64)
    z2 = _tc_layer2(agg1, z1, norm, W2, b1.reshape(1, H))
    agg2 = _sc_segsum(src1d, dst1d, z2, zeros64)

    ridx = reachable_indices.astype(jnp.int32)
    cidx = jnp.asarray(current_node_idx, jnp.int32).reshape(1)
    eidx = neighbor_edge_idx.astype(jnp.int32)
    wa = Wm1[:H]
    wb = Wm1[H:2 * H]
    wc = Wm1[2 * H:]
    q = _tc_final(agg2, z2, norm, b2.reshape(1, H), wa, wb, wc,
                  bm1.reshape(1, 2 * H), Wm2.reshape(1, 2 * H),
                  bm2.reshape(1, 1), ridx, cidx, eidx, edge_attr)
    return q.reshape(K)


# speedup vs baseline: 28.8877x; 1.0702x over previous
import functools

import jax
import jax.numpy as jnp
from jax import lax
from jax.experimental import pallas as pl
from jax.experimental.pallas import tpu as pltpu
from jax.experimental.pallas import tpu_sc as plsc

N = 10000
NP = 10240
E = 320000
D = 128
H = 64
DE = 16
K = 64

NC = 2
NS = 16
NW = NC * NS
EPW = E // NW
CH = 400
NCH = EPW // CH
CHD = 2000
NCHD = EPW // CHD
ROWS_PER_TILE = NP // NS

_mesh = plsc.VectorSubcoreMesh(
    core_axis_name="c", subcore_axis_name="s", num_cores=NC, num_subcores=NS)


@functools.partial(
    pl.kernel,
    out_type=jax.ShapeDtypeStruct((NC, NP, 16), jnp.float32),
    mesh=_mesh,
    compiler_params=pltpu.CompilerParams(use_tc_tiling_on_sc=False),
    scratch_types=[
        pltpu.VMEM((EPW,), jnp.int32),
        pltpu.VMEM((CHD, 16), jnp.float32),
        pltpu.VMEM_SHARED((NP, 16), jnp.float32),
        pltpu.SemaphoreType.DMA,
        pltpu.SemaphoreType.DMA,
    ],
)
def _sc_degree(dst1d, ones_t, zeros16, out, dstv, onesv, acc, semi, sems):
    c = lax.axis_index("c")
    s = lax.axis_index("s")
    g = c * NS + s

    ci = pltpu.async_copy(dst1d.at[pl.ds(g * EPW, EPW)], dstv, semi)
    pltpu.sync_copy(zeros16, acc.at[pl.ds(s * ROWS_PER_TILE, ROWS_PER_TILE)])
    pltpu.sync_copy(ones_t, onesv)
    ci.wait()
    plsc.subcore_barrier()

    descs = [pltpu.async_copy(onesv, acc.at[dstv.at[pl.ds(b * CHD, CHD)]],
                              sems, add=True)
             for b in range(NCHD)]
    for d in descs:
        d.wait()

    plsc.subcore_barrier()
    pltpu.sync_copy(acc.at[pl.ds(s * ROWS_PER_TILE, ROWS_PER_TILE)],
                    out.at[c, pl.ds(s * ROWS_PER_TILE, ROWS_PER_TILE)])


@functools.partial(
    pl.kernel,
    out_type=jax.ShapeDtypeStruct((NC, NP, H), jnp.float32),
    mesh=_mesh,
    compiler_params=pltpu.CompilerParams(use_tc_tiling_on_sc=False),
    scratch_types=[
        pltpu.VMEM((EPW,), jnp.int32),
        pltpu.VMEM((EPW,), jnp.int32),
        pltpu.VMEM((CH, H), jnp.float32),
        pltpu.VMEM((CH, H), jnp.float32),
        pltpu.VMEM_SHARED((NP, H), jnp.float32),
        pltpu.SemaphoreType.DMA,
        pltpu.SemaphoreType.DMA,
        pltpu.SemaphoreType.DMA,
        pltpu.SemaphoreType.DMA,
        pltpu.SemaphoreType.DMA,
    ],
)
def _sc_segsum(src1d, dst1d, table, zeros64, out, srcv, dstv, rb0, rb1, acc,
               semi, semg0, semg1, sems0, sems1):
    c = lax.axis_index("c")
    s = lax.axis_index("s")
    g = c * NS + s
    e0 = g * EPW

    ci = pltpu.async_copy(src1d.at[pl.ds(e0, EPW)], srcv, semi)
    cd = pltpu.async_copy(dst1d.at[pl.ds(e0, EPW)], dstv, semi)
    pltpu.sync_copy(zeros64, acc.at[pl.ds(s * ROWS_PER_TILE, ROWS_PER_TILE)])
    ci.wait()
    cd.wait()
    plsc.subcore_barrier()

    rbs = (rb0, rb1)
    semg = (semg0, semg1)
    sems = (sems0, sems1)
    gd = [None, None]
    sd = [None, None]
    gd[0] = pltpu.async_copy(table.at[srcv.at[pl.ds(0, CH)]], rbs[0], semg[0])
    for b in range(NCH):
        sl = b % 2
        gd[sl].wait()
        sd[sl] = pltpu.async_copy(rbs[sl], acc.at[dstv.at[pl.ds(b * CH, CH)]],
                                  sems[sl], add=True)
        if b + 1 < NCH:
            ns = (b + 1) % 2
            if sd[ns] is not None:
                sd[ns].wait()
            gd[ns] = pltpu.async_copy(
                table.at[srcv.at[pl.ds((b + 1) * CH, CH)]], rbs[ns], semg[ns])
    sd[(NCH - 2) % 2].wait()
    sd[(NCH - 1) % 2].wait()

    plsc.subcore_barrier()
    pltpu.sync_copy(acc.at[pl.ds(s * ROWS_PER_TILE, ROWS_PER_TILE)],
                    out.at[c, pl.ds(s * ROWS_PER_TILE, ROWS_PER_TILE)])


NSLOT = 96
CHP = 400

@functools.partial(
    pl.kernel,
    out_type=jax.ShapeDtypeStruct((NW, NSLOT, H), jnp.float32),
    mesh=_mesh,
    compiler_params=pltpu.CompilerParams(use_tc_tiling_on_sc=False,
                                         needs_layout_passes=False),
    scratch_types=[
        pltpu.VMEM((EPW,), jnp.int32),
        pltpu.VMEM((EPW,), jnp.int32),
        pltpu.VMEM((NP,), jnp.int32),
        pltpu.VMEM((EPW + 16,), jnp.int32),
        pltpu.VMEM((EPW + 16,), jnp.int32),
        pltpu.VMEM((CHP, H), jnp.float32),
        pltpu.VMEM_SHARED((NS * NSLOT, H), jnp.float32),
        pltpu.SemaphoreType.DMA,
        pltpu.SemaphoreType.DMA,
    ],
)
def _sc_segsum_pruned(src1d, dst1d, table, m_in, out, srcv, dstv, m_ref,
                      csrc, cslot, rowbuf, acc, semi, semg):
    c = lax.axis_index("c")
    s = lax.axis_index("s")
    g = c * NS + s
    e0 = g * EPW

    ci = pltpu.async_copy(src1d.at[pl.ds(e0, EPW)], srcv, semi)
    cd = pltpu.async_copy(dst1d.at[pl.ds(e0, EPW)], dstv, semi)
    cm = pltpu.async_copy(m_in, m_ref, semi)

    lane = lax.iota(jnp.int32, 16)

    soff = s * NSLOT

    def fill_pads(i, _):
        base = 10016 + lax.rem(i, 14) * 16
        csrc[pl.ds(i * 16, 16)] = base + lane
        cslot[pl.ds(i * 16, 16)] = soff + 65 + lane
        return 0
    lax.fori_loop(0, (EPW + 16) // 16, fill_pads, 0)

    def zero_rb(i, _):
        r = i // (H // 16)
        col = lax.rem(i, H // 16)
        rowbuf[r, pl.ds(col * 16, 16)] = jnp.zeros((16,), jnp.float32)
        return 0
    lax.fori_loop(0, NSLOT * (H // 16), zero_rb, 0)
    pltpu.sync_copy(rowbuf.at[pl.ds(0, NSLOT)], acc.at[pl.ds(soff, NSLOT)])

    ci.wait()
    cd.wait()
    cm.wait()

    def compact(i, off):
        dvec = dstv[pl.ds(i * 16, 16)]
        slots = plsc.load_gather(m_ref, [dvec])
        mask = slots < 65
        svec = srcv[pl.ds(i * 16, 16)]
        plsc.store_compressed(csrc.at[pl.ds(off, 16)], svec, mask=mask)
        plsc.store_compressed(cslot.at[pl.ds(off, 16)], slots + soff, mask=mask)
        return off + jnp.sum(mask.astype(jnp.int32))
    cnt = lax.fori_loop(0, EPW // 16, compact, jnp.int32(0))

    nchunks = (cnt + CHP - 1) // CHP

    def drain(t, _):
        pltpu.async_copy(table.at[csrc.at[pl.ds(t * CHP, CHP)]],
                         rowbuf, semg).wait()
        pltpu.sync_copy(rowbuf, acc.at[cslot.at[pl.ds(t * CHP, CHP)]],
                        add=True)
        return 0
    lax.fori_loop(0, nchunks, drain, 0)

    pltpu.sync_copy(acc.at[pl.ds(soff, NSLOT)], out.at[g])


def _tc_edges_body(ei_ref, rl_ref, src_ref, dst_ref, m_ref):
    src_ref[...] = ei_ref[0]
    dst_ref[...] = ei_ref[1]
    m_ref[...] = jnp.full((NP, 1), 65, jnp.int32)

    def set_m(j, _):
        jj = 64 - j
        m_ref[pl.ds(rl_ref[jj], 1), :] = jnp.full((1, 1), jj, jnp.int32)
        return 0
    lax.fori_loop(0, 65, set_m, 0)


def _tc_edges(ei, rl):
    return pl.pallas_call(
        _tc_edges_body,
        in_specs=[pl.BlockSpec(memory_space=pltpu.VMEM),
                  pl.BlockSpec(memory_space=pltpu.SMEM)],
        out_shape=(jax.ShapeDtypeStruct((E,), jnp.int32),
                   jax.ShapeDtypeStruct((E,), jnp.int32),
                   jax.ShapeDtypeStruct((NP, 1), jnp.int32)),
    )(ei, rl)


def _tc_layer1_body(x_ref, w1_ref, degp_ref, z1_ref, norm_ref):
    deg = degp_ref[0, :, 0:1] + degp_ref[1, :, 0:1] + 1.0
    norm = lax.rsqrt(deg)
    u = jnp.dot(x_ref[...], w1_ref[...], preferred_element_type=jnp.float32)
    z1_ref[0:N] = u * norm[0:N]
    z1_ref[N:NP] = jnp.zeros((NP - N, H), jnp.float32)
    norm_ref[...] = norm


def _tc_layer1(x, w1, degparts):
    return pl.pallas_call(
        _tc_layer1_body,
        out_shape=(jax.ShapeDtypeStruct((NP, H), jnp.float32),
                   jax.ShapeDtypeStruct((NP, 1), jnp.float32)),
    )(x, w1, degparts)


def _tc_layer2_body(aggp_ref, z1_ref, norm_ref, w2_ref, b1_ref, z2_ref):
    norm = norm_ref[...]
    ssum = aggp_ref[0] + aggp_ref[1] + z1_ref[...]
    h = jnp.maximum(norm * ssum + b1_ref[...], 0.0)
    z2 = jnp.dot(h, w2_ref[...], preferred_element_type=jnp.float32) * norm
    z2_ref[0:N] = z2[0:N]
    z2_ref[N:NP] = jnp.zeros((NP - N, H), jnp.float32)


def _tc_layer2(aggparts, z1, norm, w2, b1r):
    return pl.pallas_call(
        _tc_layer2_body,
        out_shape=jax.ShapeDtypeStruct((NP, H), jnp.float32),
    )(aggparts, z1, norm, w2, b1r)


def _tc_final_body(aggp_ref, z2_ref, norm_ref, b2_ref, wa_ref, wb_ref,
                   wc_ref, bm1_ref, wm2_ref, bm2_ref, ridx_ref, canon_ref,
                   eidx_ref, ea_hbm, q_ref, red_ref, hn_ref, hc_ref, ea_ref,
                   sem):
    red_ref[...] = jnp.sum(aggp_ref[...], axis=0)

    def h2row(k):
        node = ridx_ref[k]
        slot = canon_ref[k]
        aggrow = red_ref[pl.ds(slot, 1), :]
        z2row = z2_ref[pl.ds(node, 1), :]
        nv = norm_ref[pl.ds(node, 1), :]
        return jnp.maximum(nv * (aggrow + z2row) + b2_ref[...], 0.0)

    def gather_body(k, _):
        hn_ref[pl.ds(k, 1), :] = h2row(k)
        eidx = eidx_ref[k]
        pltpu.make_async_copy(ea_hbm.at[pl.ds(eidx, 1)],
                              ea_ref.at[pl.ds(k, 1)], sem).start()
        return 0
    lax.fori_loop(0, K, gather_body, 0)
    hc_ref[...] = h2row(K)

    def drain_body(k, _):
        pltpu.make_async_copy(ea_hbm.at[pl.ds(0, 1)],
                              ea_ref.at[pl.ds(0, 1)], sem).wait()
        return 0
    lax.fori_loop(0, K, drain_body, 0)

    acc = jnp.dot(hc_ref[...], wa_ref[...], preferred_element_type=jnp.float32)
    acc = acc + jnp.dot(hn_ref[...], wb_ref[...],
                        preferred_element_type=jnp.float32)
    acc = acc + jnp.dot(ea_ref[...], wc_ref[...],
                        preferred_element_type=jnp.float32)
    pre = jnp.maximum(acc + bm1_ref[...], 0.0)
    q = jnp.sum(pre * wm2_ref[...], axis=1, keepdims=True) + bm2_ref[...]
    q_ref[...] = q


def _tc_final(aggparts, z2, norm, b2r, wa, wb, wc, bm1r, wm2r, bm2r,
              ridx, canon, eidx, edge_attr):
    return pl.pallas_call(
        _tc_final_body,
        out_shape=jax.ShapeDtypeStruct((K, 1), jnp.float32),
        in_specs=[pl.BlockSpec(memory_space=pl.ANY)
                  if i == 13 else
                  (pl.BlockSpec(memory_space=pltpu.SMEM)
                   if i in (10, 11, 12) else pl.BlockSpec(memory_space=pltpu.VMEM))
                  for i in range(14)],
        out_specs=pl.BlockSpec(memory_space=pltpu.VMEM),
        scratch_shapes=[
            pltpu.VMEM((NSLOT, H), jnp.float32),
            pltpu.VMEM((K, H), jnp.float32),
            pltpu.VMEM((1, H), jnp.float32),
            pltpu.VMEM((K, DE), jnp.float32),
            pltpu.SemaphoreType.DMA,
        ],
    )(aggparts, z2, norm, b2r, wa, wb, wc, bm1r, wm2r, bm2r,
      ridx, canon, eidx, edge_attr)


def kernel(x, edge_index, edge_attr, current_node_idx, reachable_indices,
           neighbor_edge_idx, W1, b1, W2, b2, Wm1, bm1, Wm2, bm2):
    ridx = reachable_indices.astype(jnp.int32)
    cidx = jnp.asarray(current_node_idx, jnp.int32).reshape(1)
    rl = jnp.concatenate([ridx, cidx])
    src1d, dst1d, m2d = _tc_edges(edge_index.astype(jnp.int32), rl)

    ones_t = jnp.ones((CHD, 16), jnp.float32)
    zeros16 = jnp.zeros((ROWS_PER_TILE, 16), jnp.float32)
    zeros64 = jnp.zeros((ROWS_PER_TILE, H), jnp.float32)

    degparts = _sc_degree(dst1d, ones_t, zeros16)
    z1, norm = _tc_layer1(x, W1, degparts)
    agg1 = _sc_segsum(src1d, dst1d, z1, zeros64)
    z2 = _tc_layer2(agg1, z1, norm, W2, b1.reshape(1, H))

    r65 = rl
    canon = jnp.argmax(r65[None, :] == r65[:, None], axis=1).astype(jnp.int32)
    agg2 = _sc_segsum_pruned(src1d, dst1d, z2, m2d.reshape(NP))

    eidx = neighbor_edge_idx.astype(jnp.int32)
    wa = Wm1[:H]
    wb = Wm1[H:2 * H]
    wc = Wm1[2 * H:]
    q = _tc_final(agg2, z2, norm, b2.reshape(1, H), wa, wb, wc,
                  bm1.reshape(1, 2 * H), Wm2.reshape(1, 2 * H),
                  bm2.reshape(1, 1), r65, canon, eidx, edge_attr)
    return q.reshape(K)
